# baseline (speedup 1.0000x reference)
import jax
import jax.numpy as jnp
from jax import lax
from jax.experimental import pallas as pl
from jax.experimental.pallas import tpu as pltpu
from jax.experimental.pallas import tpu_sc as plsc

B = 16384
D = 32
NC = 2
NS = 16
L = 16
NW = NC * NS
BPW = B // NW


def _sc_kernel(users_hbm, items_hbm, utab_hbm, itab_hbm, out_hbm,
               uidx_v, iidx_v, ug_v, ig_v, out_v, sem):
    wid = lax.axis_index("s") * NC + lax.axis_index("c")
    base = wid * BPW
    pltpu.sync_copy(users_hbm.at[pl.ds(base, BPW)], uidx_v)
    pltpu.sync_copy(items_hbm.at[pl.ds(base, BPW)], iidx_v)

    copies = []
    for d in range(D):
        copies.append(pltpu.async_copy(
            utab_hbm.at[d].at[plsc.Indices(uidx_v)], ug_v.at[d], sem))
        copies.append(pltpu.async_copy(
            itab_hbm.at[d].at[plsc.Indices(iidx_v)], ig_v.at[d], sem))
    for c in copies:
        c.wait()

    @pl.loop(0, BPW // L)
    def _(g):
        sl = pl.ds(g * L, L)
        acc = jnp.zeros((L,), jnp.float32)
        for d in range(D):
            acc = acc + ug_v[d, sl] * ig_v[d, sl]
        out_v[sl] = 1.0 / (1.0 + jnp.exp(-acc))

    pltpu.sync_copy(out_v, out_hbm.at[pl.ds(base, BPW)])


@jax.jit
def _run(users, items, user_table, item_table):
    mesh = plsc.VectorSubcoreMesh(core_axis_name="c", subcore_axis_name="s")
    cp = pltpu.CompilerParams(
        needs_layout_passes=False, use_tc_tiling_on_sc=False)
    k = pl.kernel(
        _sc_kernel,
        out_type=jax.ShapeDtypeStruct((B,), jnp.float32),
        mesh=mesh,
        scratch_types=[
            pltpu.VMEM((BPW,), jnp.int32),
            pltpu.VMEM((BPW,), jnp.int32),
            pltpu.VMEM((D, BPW), jnp.float32),
            pltpu.VMEM((D, BPW), jnp.float32),
            pltpu.VMEM((BPW,), jnp.float32),
            pltpu.SemaphoreType.DMA,
        ],
        compiler_params=cp,
    )
    return k(users, items, user_table.T, item_table.T)


def kernel(users, items, user_table, item_table):
    return _run(users, items, user_table, item_table)


# speedup vs baseline: 20.4527x; 20.4527x over previous
import jax
import jax.numpy as jnp
from jax import lax
from jax.experimental import pallas as pl
from jax.experimental.pallas import tpu as pltpu
from jax.experimental.pallas import tpu_sc as plsc

B = 16384
D = 32
V = 1000000
NC = 2
NS = 16
L = 16
NW = NC * NS
BPW = B // NW
OCT = 8
NOCT = BPW // OCT

IDXPAD = BPW + L


def _sc_kernel(users_hbm, items_hbm, utab_hbm, itab_hbm, out_hbm,
               uidx_v, iidx_v, ubuf_v, ibuf_v, out_v, sem):
    lanes = lax.iota(jnp.int32, L)
    a_lo = lanes >> 3
    a_hi = a_lo + 2
    k_sel = lanes & 7
    wid = lax.axis_index("s") * NC + lax.axis_index("c")
    base = wid * BPW
    pltpu.sync_copy(users_hbm.at[pl.ds(base, BPW)],
                    uidx_v.at[pl.ds(0, BPW)])
    pltpu.sync_copy(items_hbm.at[pl.ds(base, BPW)],
                    iidx_v.at[pl.ds(0, BPW)])

    @pl.loop(0, NOCT)
    def _(o):
        uvec = uidx_v[pl.ds(o * OCT, L)]
        ivec = iidx_v[pl.ds(o * OCT, L)]
        copies = []
        for j in range(OCT):
            qu = pl.multiple_of((uvec[j] >> 7) << 7, 128)
            qi = pl.multiple_of((ivec[j] >> 7) << 7, 128)
            copies.append(pltpu.async_copy(
                utab_hbm.at[:, :, pl.ds(qu, 128)], ubuf_v.at[j], sem))
            copies.append(pltpu.async_copy(
                itab_hbm.at[:, :, pl.ds(qi, 128)], ibuf_v.at[j], sem))
        for c in copies:
            c.wait()

        res = jnp.zeros((L,), jnp.float32)
        for j in range(OCT):
            mu = jnp.full((L,), uvec[j] & 127, jnp.int32)
            mi = jnp.full((L,), ivec[j] & 127, jnp.int32)
            uv0 = plsc.load_gather(ubuf_v.at[j], [a_lo, k_sel, mu])
            uv1 = plsc.load_gather(ubuf_v.at[j], [a_hi, k_sel, mu])
            iv0 = plsc.load_gather(ibuf_v.at[j], [a_lo, k_sel, mi])
            iv1 = plsc.load_gather(ibuf_v.at[j], [a_hi, k_sel, mi])
            s = jnp.sum(uv0 * iv0 + uv1 * iv1)
            res = jnp.where(lanes == j, s, res)
        out_v[pl.ds(o * OCT, L)] = res

    @pl.loop(0, BPW // L)
    def _(g):
        sl = pl.ds(g * L, L)
        x = out_v[sl]
        out_v[sl] = 1.0 / (1.0 + jnp.exp(-x))

    pltpu.sync_copy(out_v.at[pl.ds(0, BPW)], out_hbm.at[pl.ds(base, BPW)])


@jax.jit
def _run(users, items, user_table, item_table):
    mesh = plsc.VectorSubcoreMesh(core_axis_name="c", subcore_axis_name="s")
    cp = pltpu.CompilerParams(
        needs_layout_passes=False, use_tc_tiling_on_sc=True)
    k = pl.kernel(
        _sc_kernel,
        out_type=jax.ShapeDtypeStruct((B,), jnp.float32),
        mesh=mesh,
        scratch_types=[
            pltpu.VMEM((IDXPAD,), jnp.int32),
            pltpu.VMEM((IDXPAD,), jnp.int32),
            pltpu.VMEM((OCT, 4, 8, 128), jnp.float32),
            pltpu.VMEM((OCT, 4, 8, 128), jnp.float32),
            pltpu.VMEM((IDXPAD,), jnp.float32),
            pltpu.SemaphoreType.DMA,
        ],
        compiler_params=cp,
    )
    ut = user_table.T.reshape(4, 8, V)
    it = item_table.T.reshape(4, 8, V)
    return k(users, items, ut, it)


def kernel(users, items, user_table, item_table):
    return _run(users, items, user_table, item_table)


# speedup vs baseline: 24.7006x; 1.2077x over previous
import jax
import jax.numpy as jnp
from jax import lax
from jax.experimental import pallas as pl
from jax.experimental.pallas import tpu as pltpu
from jax.experimental.pallas import tpu_sc as plsc

B = 16384
D = 32
V = 1000000
NC = 2
NS = 16
L = 16
NW = NC * NS
BPW = B // NW
OCT = 8
NOCT = BPW // OCT

IDXPAD = BPW + L


def _sc_kernel(users_hbm, items_hbm, utab_hbm, itab_hbm, out_hbm,
               uidx_v, iidx_v, ubuf_v, ibuf_v, out_v,
               sem0, sem1, sem2, sem3, sem4, sem5, sem6, sem7):
    sems = [sem0, sem1, sem2, sem3, sem4, sem5, sem6, sem7]
    lanes = lax.iota(jnp.int32, L)
    a_lo = lanes >> 3
    a_hi = a_lo + 2
    k_sel = lanes & 7
    wid = lax.axis_index("s") * NC + lax.axis_index("c")
    base = wid * BPW
    pltpu.sync_copy(users_hbm.at[pl.ds(base, BPW)],
                    uidx_v.at[pl.ds(0, BPW)])
    pltpu.sync_copy(items_hbm.at[pl.ds(base, BPW)],
                    iidx_v.at[pl.ds(0, BPW)])

    def fire(j, ru, ri):
        qu = pl.multiple_of((ru >> 7) << 7, 128)
        qi = pl.multiple_of((ri >> 7) << 7, 128)
        pltpu.async_copy(
            utab_hbm.at[:, :, pl.ds(qu, 128)], ubuf_v.at[j], sems[j])
        pltpu.async_copy(
            itab_hbm.at[:, :, pl.ds(qi, 128)], ibuf_v.at[j], sems[j])

    def wait_slot(j):
        dummy = utab_hbm.at[:, :, pl.ds(0, 128)]
        pltpu.make_async_copy(dummy, ubuf_v.at[j], sems[j]).wait()
        pltpu.make_async_copy(dummy, ibuf_v.at[j], sems[j]).wait()

    uvec0 = uidx_v[pl.ds(0, L)]
    ivec0 = iidx_v[pl.ds(0, L)]
    for j in range(OCT):
        fire(j, uvec0[j], ivec0[j])

    @pl.loop(0, NOCT)
    def _(o):
        uvec = uidx_v[pl.ds(o * OCT, L)]
        ivec = iidx_v[pl.ds(o * OCT, L)]
        uvn = uidx_v[pl.ds(o * OCT + OCT, L)]
        ivn = iidx_v[pl.ds(o * OCT + OCT, L)]

        res = jnp.zeros((L,), jnp.float32)
        for j in range(OCT):
            wait_slot(j)
            mu = jnp.full((L,), uvec[j] & 127, jnp.int32)
            mi = jnp.full((L,), ivec[j] & 127, jnp.int32)
            uv0 = plsc.load_gather(ubuf_v.at[j], [a_lo, k_sel, mu])
            uv1 = plsc.load_gather(ubuf_v.at[j], [a_hi, k_sel, mu])
            iv0 = plsc.load_gather(ibuf_v.at[j], [a_lo, k_sel, mi])
            iv1 = plsc.load_gather(ibuf_v.at[j], [a_hi, k_sel, mi])
            s = jnp.sum(uv0 * iv0 + uv1 * iv1)
            res = jnp.where(lanes == j, s, res)

            @pl.when(o + 1 < NOCT)
            def _():
                fire(j, uvn[j], ivn[j])

        out_v[pl.ds(o * OCT, L)] = res

    @pl.loop(0, BPW // L)
    def _(g):
        sl = pl.ds(g * L, L)
        x = out_v[sl]
        out_v[sl] = 1.0 / (1.0 + jnp.exp(-x))

    pltpu.sync_copy(out_v.at[pl.ds(0, BPW)], out_hbm.at[pl.ds(base, BPW)])


@jax.jit
def _run(users, items, user_table, item_table):
    mesh = plsc.VectorSubcoreMesh(core_axis_name="c", subcore_axis_name="s")
    cp = pltpu.CompilerParams(
        needs_layout_passes=False, use_tc_tiling_on_sc=True)
    k = pl.kernel(
        _sc_kernel,
        out_type=jax.ShapeDtypeStruct((B,), jnp.float32),
        mesh=mesh,
        scratch_types=[
            pltpu.VMEM((IDXPAD,), jnp.int32),
            pltpu.VMEM((IDXPAD,), jnp.int32),
            pltpu.VMEM((OCT, 4, 8, 128), jnp.float32),
            pltpu.VMEM((OCT, 4, 8, 128), jnp.float32),
            pltpu.VMEM((IDXPAD,), jnp.float32),
        ] + [pltpu.SemaphoreType.DMA] * OCT,
        compiler_params=cp,
    )
    ut = user_table.T.reshape(4, 8, V)
    it = item_table.T.reshape(4, 8, V)
    return k(users, items, ut, it)


def kernel(users, items, user_table, item_table):
    return _run(users, items, user_table, item_table)


# speedup vs baseline: 24.8077x; 1.0043x over previous
import jax
import jax.numpy as jnp
from jax import lax
from jax.experimental import pallas as pl
from jax.experimental.pallas import tpu as pltpu
from jax.experimental.pallas import tpu_sc as plsc

B = 16384
D = 32
V = 1000000
NC = 2
NS = 16
L = 16
NW = NC * NS
BPW = B // NW
OCT = 8
NOCT = BPW // OCT

IDXPAD = BPW + L


def _sc_kernel(users_hbm, items_hbm, utab_hbm, itab_hbm, out_hbm,
               uidx_v, iidx_v, ubuf_v, ibuf_v, out_v,
               sem0, sem1, sem2, sem3, sem4, sem5, sem6, sem7):
    sems = [sem0, sem1, sem2, sem3, sem4, sem5, sem6, sem7]
    lanes = lax.iota(jnp.int32, L)
    a_lo = lanes >> 3
    a_hi = a_lo + 2
    k_sel = lanes & 7
    wid = lax.axis_index("s") * NC + lax.axis_index("c")
    base = wid * BPW
    pltpu.sync_copy(users_hbm.at[pl.ds(base, BPW)],
                    uidx_v.at[pl.ds(0, BPW)])
    pltpu.sync_copy(items_hbm.at[pl.ds(base, BPW)],
                    iidx_v.at[pl.ds(0, BPW)])

    def fire(j, ru, ri):
        qu = pl.multiple_of((ru >> 7) << 7, 128)
        qi = pl.multiple_of((ri >> 7) << 7, 128)
        for a in range(4):
            pltpu.async_copy(
                utab_hbm.at[pl.ds(a, 1), :, pl.ds(qu, 128)],
                ubuf_v.at[j].at[pl.ds(a, 1)], sems[j])
            pltpu.async_copy(
                itab_hbm.at[pl.ds(a, 1), :, pl.ds(qi, 128)],
                ibuf_v.at[j].at[pl.ds(a, 1)], sems[j])

    def wait_slot(j):
        dummy = utab_hbm.at[:, :, pl.ds(0, 128)]
        pltpu.make_async_copy(dummy, ubuf_v.at[j], sems[j]).wait()
        pltpu.make_async_copy(dummy, ibuf_v.at[j], sems[j]).wait()

    uvec0 = uidx_v[pl.ds(0, L)]
    ivec0 = iidx_v[pl.ds(0, L)]
    for j in range(OCT):
        fire(j, uvec0[j], ivec0[j])

    @pl.loop(0, NOCT)
    def _(o):
        uvec = uidx_v[pl.ds(o * OCT, L)]
        ivec = iidx_v[pl.ds(o * OCT, L)]
        uvn = uidx_v[pl.ds(o * OCT + OCT, L)]
        ivn = iidx_v[pl.ds(o * OCT + OCT, L)]

        res = jnp.zeros((L,), jnp.float32)
        for j in range(OCT):
            wait_slot(j)
            mu = jnp.full((L,), uvec[j] & 127, jnp.int32)
            mi = jnp.full((L,), ivec[j] & 127, jnp.int32)
            uv0 = plsc.load_gather(ubuf_v.at[j], [a_lo, k_sel, mu])
            uv1 = plsc.load_gather(ubuf_v.at[j], [a_hi, k_sel, mu])
            iv0 = plsc.load_gather(ibuf_v.at[j], [a_lo, k_sel, mi])
            iv1 = plsc.load_gather(ibuf_v.at[j], [a_hi, k_sel, mi])
            s = jnp.sum(uv0 * iv0 + uv1 * iv1)
            res = jnp.where(lanes == j, s, res)

            @pl.when(o + 1 < NOCT)
            def _():
                fire(j, uvn[j], ivn[j])

        out_v[pl.ds(o * OCT, L)] = res

    @pl.loop(0, BPW // L)
    def _(g):
        sl = pl.ds(g * L, L)
        x = out_v[sl]
        out_v[sl] = 1.0 / (1.0 + jnp.exp(-x))

    pltpu.sync_copy(out_v.at[pl.ds(0, BPW)], out_hbm.at[pl.ds(base, BPW)])


@jax.jit
def _run(users, items, user_table, item_table):
    mesh = plsc.VectorSubcoreMesh(core_axis_name="c", subcore_axis_name="s")
    cp = pltpu.CompilerParams(
        needs_layout_passes=False, use_tc_tiling_on_sc=True)
    k = pl.kernel(
        _sc_kernel,
        out_type=jax.ShapeDtypeStruct((B,), jnp.float32),
        mesh=mesh,
        scratch_types=[
            pltpu.VMEM((IDXPAD,), jnp.int32),
            pltpu.VMEM((IDXPAD,), jnp.int32),
            pltpu.VMEM((OCT, 4, 8, 128), jnp.float32),
            pltpu.VMEM((OCT, 4, 8, 128), jnp.float32),
            pltpu.VMEM((IDXPAD,), jnp.float32),
        ] + [pltpu.SemaphoreType.DMA] * OCT,
        compiler_params=cp,
    )
    ut = user_table.T.reshape(4, 8, V)
    it = item_table.T.reshape(4, 8, V)
    return k(users, items, ut, it)


def kernel(users, items, user_table, item_table):
    return _run(users, items, user_table, item_table)
